# Initial kernel scaffold; baseline (speedup 1.0000x reference)
import jax
import jax.numpy as jnp
from jax import lax
from jax.experimental import pallas as pl
from jax.experimental.pallas import tpu as pltpu
from jax.experimental.pallas import tpu_sc as plsc

_NUM_FIELDS = 26
_FIELD_DIM = 100000
_BATCH = 16384
_NC = 2
_NS = 16
_NW = _NC * _NS
_L = 16
_BPW = _BATCH // _NW
_IPW = _BPW * _NUM_FIELDS
_NROW = _IPW // 128


def _body(x_hbm, w_hbm, bias_hbm, out_hbm, xv, idx2, vals, acc, bias_v, sem):
    wid = lax.axis_index("s") * _NC + lax.axis_index("c")
    base = wid * _IPW

    pltpu.sync_copy(x_hbm.at[pl.ds(base, _IPW)], xv)
    pltpu.sync_copy(bias_hbm, bias_v)
    bias_vec = bias_v[...]

    iota26 = lax.broadcasted_iota(jnp.int32, (_L,), 0) * _NUM_FIELDS

    for f in range(_NUM_FIELDS):
        off = f * _FIELD_DIM
        for j in range(_BPW // _L):
            pos = iota26 + (_NUM_FIELDS * _L * j + f)
            xg = plsc.load_gather(xv, [pos])
            q = f * _BPW + _L * j
            idx2[q // 128, pl.ds(q % 128, _L)] = xg + off

    handles = [
        pltpu.async_copy(w_hbm.at[idx2.at[r]], vals.at[r], sem)
        for r in range(_NROW)
    ]
    for h in handles:
        h.wait()

    for j in range(_BPW // _L):
        a = bias_vec
        for f in range(_NUM_FIELDS):
            q = f * _BPW + _L * j
            a = a + vals[q // 128, pl.ds(q % 128, _L)]
        acc[pl.ds(_L * j, _L)] = a

    pltpu.sync_copy(acc, out_hbm.at[pl.ds(wid * _BPW, _BPW)])


def kernel(x, W, bias):
    x_flat = x.reshape(-1)
    w_flat = W.reshape(-1)
    bias16 = jnp.broadcast_to(bias, (_L,)).astype(jnp.float32)

    mesh = plsc.VectorSubcoreMesh(core_axis_name="c", subcore_axis_name="s")
    fn = pl.kernel(
        _body,
        out_type=jax.ShapeDtypeStruct((_BATCH,), jnp.float32),
        mesh=mesh,
        scratch_types=[
            pltpu.VMEM((_IPW,), jnp.int32),
            pltpu.VMEM((_NROW, 128), jnp.int32),
            pltpu.VMEM((_NROW, 128), jnp.float32),
            pltpu.VMEM((_BPW,), jnp.float32),
            pltpu.VMEM((_L,), jnp.float32),
            pltpu.SemaphoreType.DMA,
        ],
    )
    return fn(x_flat, w_flat, bias16)


# speedup vs baseline: 1.0852x; 1.0852x over previous
import jax
import jax.numpy as jnp
from jax import lax
from jax.experimental import pallas as pl
from jax.experimental.pallas import tpu as pltpu
from jax.experimental.pallas import tpu_sc as plsc

_NUM_FIELDS = 26
_FIELD_DIM = 100000
_BATCH = 16384
_NC = 2
_NS = 16
_NW = _NC * _NS
_L = 16
_BPW = _BATCH // _NW
_IPW = _BPW * _NUM_FIELDS
_NROW = _IPW // 128


def _body(x_hbm, w_hbm, bias_hbm, out_hbm, xv, idx2, vals, acc, bias_v, sem):
    wid = lax.axis_index("s") * _NC + lax.axis_index("c")
    base = wid * _IPW

    pltpu.sync_copy(x_hbm.at[pl.ds(base, _IPW)], xv)
    pltpu.sync_copy(bias_hbm, bias_v)
    bias_vec = bias_v[...]

    iota26 = lax.broadcasted_iota(jnp.int32, (_L,), 0) * _NUM_FIELDS

    for f in range(_NUM_FIELDS):
        off = f * _FIELD_DIM
        for j in range(_BPW // _L):
            pos = iota26 + (_NUM_FIELDS * _L * j + f)
            xg = plsc.load_gather(xv, [pos])
            q = f * _BPW + _L * j
            idx2[q // 128, pl.ds(q % 128, _L)] = xg + off

    handles = [
        pltpu.async_copy(w_hbm.at[idx2.at[r]], vals.at[r], sem)
        for r in range(_NROW)
    ]
    for h in handles:
        h.wait()

    for j in range(_BPW // _L):
        a = bias_vec
        for f in range(_NUM_FIELDS):
            q = f * _BPW + _L * j
            a = a + vals[q // 128, pl.ds(q % 128, _L)]
        acc[pl.ds(_L * j, _L)] = a

    pltpu.sync_copy(acc, out_hbm.at[pl.ds(wid * _BPW, _BPW)])


def kernel(x, W, bias):
    x_flat = x.reshape(-1)
    w_flat = W.reshape(-1)
    bias16 = jnp.broadcast_to(bias, (_L,)).astype(jnp.float32)

    mesh = plsc.VectorSubcoreMesh(
        core_axis_name="c", subcore_axis_name="s",
        num_cores=_NC, num_subcores=_NS,
    )
    fn = pl.kernel(
        _body,
        out_type=jax.ShapeDtypeStruct((_BATCH,), jnp.float32),
        mesh=mesh,
        compiler_params=pltpu.CompilerParams(needs_layout_passes=False),
        scratch_types=[
            pltpu.VMEM((_IPW,), jnp.int32),
            pltpu.VMEM((_NROW, 128), jnp.int32),
            pltpu.VMEM((_NROW, 128), jnp.float32),
            pltpu.VMEM((_BPW,), jnp.float32),
            pltpu.VMEM((_L,), jnp.float32),
            pltpu.SemaphoreType.DMA,
        ],
    )
    return fn(x_flat, w_flat, bias16)


# speedup vs baseline: 1.1026x; 1.0160x over previous
import jax
import jax.numpy as jnp
from jax import lax
from jax.experimental import pallas as pl
from jax.experimental.pallas import tpu as pltpu
from jax.experimental.pallas import tpu_sc as plsc

_NUM_FIELDS = 26
_FIELD_DIM = 100000
_BATCH = 16384
_NC = 2
_NS = 16
_NW = _NC * _NS
_L = 16
_BPW = _BATCH // _NW
_IPW = _BPW * _NUM_FIELDS
_NROW = _IPW // 128
_RPF = _BPW // 128


def _body(x_hbm, w_hbm, bias_hbm, out_hbm, xv, idx2, vals, acc, bias_v, sem):
    wid = lax.axis_index("s") * _NC + lax.axis_index("c")
    base = wid * _IPW

    pltpu.sync_copy(x_hbm.at[pl.ds(base, _IPW)], xv)
    pltpu.sync_copy(bias_hbm, bias_v)
    bias_vec = bias_v[...]

    for j in range(_BPW // _L):
        acc[pl.ds(_L * j, _L)] = bias_vec

    iota26 = lax.broadcasted_iota(jnp.int32, (_L,), 0) * _NUM_FIELDS

    handles = []
    for f in range(_NUM_FIELDS):
        off = f * _FIELD_DIM
        for j in range(_BPW // _L):
            pos = iota26 + (_NUM_FIELDS * _L * j + f)
            xg = plsc.load_gather(xv, [pos])
            q = f * _BPW + _L * j
            idx2[q // 128, pl.ds(q % 128, _L)] = xg + off
        for r in range(f * _RPF, (f + 1) * _RPF):
            handles.append(
                pltpu.async_copy(w_hbm.at[idx2.at[r]], vals.at[r], sem)
            )

    for r in range(_NROW):
        handles[r].wait()
        jbase = (r % _RPF) * 8
        for c in range(8):
            j = jbase + c
            a = acc[pl.ds(_L * j, _L)] + vals[r, pl.ds(_L * c, _L)]
            acc[pl.ds(_L * j, _L)] = a

    pltpu.sync_copy(acc, out_hbm.at[pl.ds(wid * _BPW, _BPW)])


def kernel(x, W, bias):
    x_flat = x.reshape(-1)
    w_flat = W.reshape(-1)
    bias16 = jnp.broadcast_to(bias, (_L,)).astype(jnp.float32)

    mesh = plsc.VectorSubcoreMesh(
        core_axis_name="c", subcore_axis_name="s",
        num_cores=_NC, num_subcores=_NS,
    )
    fn = pl.kernel(
        _body,
        out_type=jax.ShapeDtypeStruct((_BATCH,), jnp.float32),
        mesh=mesh,
        compiler_params=pltpu.CompilerParams(needs_layout_passes=False),
        scratch_types=[
            pltpu.VMEM((_IPW,), jnp.int32),
            pltpu.VMEM((_NROW, 128), jnp.int32),
            pltpu.VMEM((_NROW, 128), jnp.float32),
            pltpu.VMEM((_BPW,), jnp.float32),
            pltpu.VMEM((_L,), jnp.float32),
            pltpu.SemaphoreType.DMA,
        ],
    )
    return fn(x_flat, w_flat, bias16)


# speedup vs baseline: 1.1547x; 1.0473x over previous
import jax
import jax.numpy as jnp
from jax import lax
from jax.experimental import pallas as pl
from jax.experimental.pallas import tpu as pltpu
from jax.experimental.pallas import tpu_sc as plsc

_NUM_FIELDS = 26
_FIELD_DIM = 100000
_BATCH = 16384
_NC = 2
_NS = 16
_L = 16
_FPC = _NUM_FIELDS // _NC
_RPT = _BATCH // _NS
_CHUNK = 4096


def _body(xt_hbm, w2_hbm, bias_hbm, out_hbm,
          tv, xi0, xi1, vchunk, spm, acc, tmp0, tmp1, bias_v, sem, sem2):
    c = lax.axis_index("c")
    s = lax.axis_index("s")
    f = s * _NC + c

    pltpu.sync_copy(bias_hbm, bias_v)

    @pl.when(s < _FPC)
    def _gather_phase():
        xis = (xi0, xi1)
        h1 = pltpu.async_copy(w2_hbm.at[f], tv, sem)
        hx = pltpu.async_copy(xt_hbm.at[f, pl.ds(0, _CHUNK)], xis[0], sem2)
        h1.wait()
        nchunk = _BATCH // _CHUNK
        for chunk in range(nchunk):
            hx.wait()
            if chunk + 1 < nchunk:
                hx = pltpu.async_copy(
                    xt_hbm.at[f, pl.ds((chunk + 1) * _CHUNK, _CHUNK)],
                    xis[(chunk + 1) % 2], sem2)
            xc = xis[chunk % 2]
            for j in range(_CHUNK // _L):
                idx = xc[pl.ds(j * _L, _L)]
                vchunk[pl.ds(j * _L, _L)] = plsc.load_gather(tv, [idx])
            pltpu.sync_copy(vchunk, spm.at[pl.ds(s * _BATCH + chunk * _CHUNK, _CHUNK)])

    plsc.subcore_barrier()

    rbase = s * _RPT
    bias_vec = bias_v[...] * (1 - c).astype(jnp.float32)
    for j in range(_RPT // _L):
        acc[pl.ds(j * _L, _L)] = bias_vec

    tmps = (tmp0, tmp1)
    h = pltpu.async_copy(spm.at[pl.ds(rbase, _RPT)], tmps[0], sem)
    for k in range(_FPC):
        h.wait()
        if k + 1 < _FPC:
            h = pltpu.async_copy(
                spm.at[pl.ds((k + 1) * _BATCH + rbase, _RPT)], tmps[(k + 1) % 2], sem)
        t = tmps[k % 2]
        for j in range(_RPT // _L):
            acc[pl.ds(j * _L, _L)] = acc[pl.ds(j * _L, _L)] + t[pl.ds(j * _L, _L)]

    pltpu.sync_copy(acc, out_hbm.at[c, pl.ds(rbase, _RPT)])


def kernel(x, W, bias):
    xt = x.T
    w2 = W.reshape(_NUM_FIELDS, _FIELD_DIM)
    bias16 = jnp.broadcast_to(bias, (_L,)).astype(jnp.float32)

    mesh = plsc.VectorSubcoreMesh(
        core_axis_name="c", subcore_axis_name="s",
        num_cores=_NC, num_subcores=_NS,
    )
    fn = pl.kernel(
        _body,
        out_type=jax.ShapeDtypeStruct((_NC, _BATCH), jnp.float32),
        mesh=mesh,
        compiler_params=pltpu.CompilerParams(needs_layout_passes=False),
        scratch_types=[
            pltpu.VMEM((_FIELD_DIM,), jnp.float32),
            pltpu.VMEM((_CHUNK,), jnp.int32),
            pltpu.VMEM((_CHUNK,), jnp.int32),
            pltpu.VMEM((_CHUNK,), jnp.float32),
            pltpu.VMEM_SHARED((_FPC * _BATCH,), jnp.float32),
            pltpu.VMEM((_RPT,), jnp.float32),
            pltpu.VMEM((_RPT,), jnp.float32),
            pltpu.VMEM((_RPT,), jnp.float32),
            pltpu.VMEM((_L,), jnp.float32),
            pltpu.SemaphoreType.DMA,
            pltpu.SemaphoreType.DMA,
        ],
    )
    partial = fn(xt, w2, bias16)
    return partial[0] + partial[1]
